# Initial kernel scaffold; baseline (speedup 1.0000x reference)
import jax
import jax.numpy as jnp
from jax.experimental import pallas as pl


def kernel(x, weights, bias, connections_index):
    raise NotImplementedError("write your pallas kernel here")


# speedup vs baseline: 41.8624x; 41.8624x over previous
import functools

import jax
import jax.numpy as jnp
from jax import lax
from jax.experimental import pallas as pl
from jax.experimental.pallas import tpu as pltpu
from jax.experimental.pallas import tpu_sc as plsc

NEURONS = 65536
FOCUS = 16
BATCH = 32
OUT_H = 256
OUT_W = 256
LANES = 16
NUM_CORES = 2
NUM_SUBCORES = 16
NW = NUM_CORES * NUM_SUBCORES
NPW = NEURONS // NW
CH = 128
NCHUNK = NPW // CH


def _make_sc_kernel():
    mesh = plsc.VectorSubcoreMesh(core_axis_name="c", subcore_axis_name="s")

    @functools.partial(
        pl.kernel,
        mesh=mesh,
        out_type=jax.ShapeDtypeStruct((NEURONS, BATCH), jnp.float32),
        scratch_types=[
            pltpu.VMEM((CH * FOCUS,), jnp.int32),
            pltpu.VMEM((CH * FOCUS, BATCH), jnp.float32),
            pltpu.VMEM((CH, FOCUS), jnp.float32),
            pltpu.VMEM((CH,), jnp.float32),
            pltpu.VMEM((CH, BATCH), jnp.float32),
            pltpu.SemaphoreType.DMA,
        ],
        compiler_params=pltpu.CompilerParams(use_tc_tiling_on_sc=False),
    )
    def sc_kernel(xT, idx, w, b, out, idxv, gv, wv, bv, ov, sem):
        wid = lax.axis_index("s") * NUM_CORES + lax.axis_index("c")
        base = wid * NPW

        def chunk_body(c, _):
            n0 = base + c * CH
            pltpu.sync_copy(idx.at[pl.ds(n0 * FOCUS, CH * FOCUS)], idxv)
            pltpu.sync_copy(w.at[pl.ds(n0, CH)], wv)
            pltpu.sync_copy(b.at[pl.ds(n0, CH)], bv)
            pltpu.async_copy(xT.at[idxv], gv, sem).wait()

            def group_body(g, _):
                g0 = g * LANES
                brow = bv[pl.ds(g0, LANES)]
                for k in range(LANES):
                    j = g0 + k
                    wrow = wv[j, :]
                    acc0 = jnp.full((LANES,), brow[k], jnp.float32)
                    acc1 = acc0
                    r = j * FOCUS
                    for f in range(FOCUS):
                        wf = jnp.full((LANES,), wrow[f], jnp.float32)
                        acc0 = acc0 + wf * gv[r + f, pl.ds(0, LANES)]
                        acc1 = acc1 + wf * gv[r + f, pl.ds(LANES, LANES)]
                    ov[j, pl.ds(0, LANES)] = acc0
                    ov[j, pl.ds(LANES, LANES)] = acc1
                return 0

            lax.fori_loop(0, CH // LANES, group_body, 0)
            pltpu.sync_copy(ov, out.at[pl.ds(n0, CH)])
            return 0

        lax.fori_loop(0, NCHUNK, chunk_body, 0)

    return sc_kernel


_SC_KERNEL = _make_sc_kernel()


def kernel(x, weights, bias, connections_index):
    batch = x.shape[0]
    xT = jnp.transpose(x.reshape(batch, -1))
    idx = connections_index.reshape(-1).astype(jnp.int32)
    outT = _SC_KERNEL(xT, idx, weights.astype(jnp.float32),
                      bias.astype(jnp.float32))
    return jnp.transpose(outT).reshape(batch, OUT_H, OUT_W)
